# baseline (speedup 1.0000x reference)
import functools

import jax
import jax.numpy as jnp
from jax import lax
from jax.experimental import pallas as pl
from jax.experimental.pallas import tpu as pltpu
from jax.experimental.pallas import tpu_sc as plsc

_NC = 2
_NS = 16
_NW = _NC * _NS

_CHG = 100
_CHS = 125
_S = 4


def _xw_kernel(x_ref, w_ref, o_ref):
    o_ref[...] = jnp.dot(x_ref[...], w_ref[...],
                         preferred_element_type=jnp.float32)


def _mlp_kernel(g_ref, dt_ref, at_ref, w1d_ref, w1a_ref, s1_ref, c1_ref,
                w2_ref, s2_ref, c2_ref, o_ref):
    d = g_ref.shape[1]
    bf = jnp.bfloat16
    cdim = (((0,), (0,)), ((), ()))
    t = (g_ref[...]
         + lax.dot_general(dt_ref[...].astype(bf), w1d_ref[...], cdim,
                           preferred_element_type=jnp.float32)
         + lax.dot_general(at_ref[...].astype(bf), w1a_ref[...], cdim,
                           preferred_element_type=jnp.float32))
    u = t * s1_ref[...] + c1_ref[...]
    u = u * jax.nn.sigmoid(u)
    v = jnp.dot(u.astype(bf), w2_ref[...], preferred_element_type=jnp.float32)
    v = v * s2_ref[...] + c2_ref[...]
    o_ref[...] = jax.nn.sigmoid(v[:, :d]) * jax.nn.softplus(v[:, d:])


def _psum_kernel(p_ref, o_ref):
    o_ref[...] = p_ref[0] + p_ref[1]


def kernel(x, edge_index, encoded_dists, encoded_angles, W1, b1, g1, bt1,
           m1, v1, W2, b2, g2, bt2, m2, v2):
    n, din = x.shape
    e = edge_index.shape[1]
    dout = W1.shape[1]
    dg = encoded_dists.shape[1]
    bf = jnp.bfloat16

    es = e // _S
    gch = es // _CHG
    cpw_g = gch // _NW
    nbg = 5
    ngr_g = cpw_g // nbg
    sch = es // _CHS
    cpw_s = sch // _NW
    nbs = 2
    ngr_s = cpw_s // nbs
    rps = n // _NS
    assert es * _S == e and gch * _CHG == es and sch * _CHS == es
    assert ngr_g * nbg == cpw_g and ngr_s * nbs == cpw_s and rps * _NS == n

    s1 = g1 * lax.rsqrt(v1 + 1e-5)
    c1 = (b1 - m1) * s1 + bt1
    s2 = g2 * lax.rsqrt(v2 + 1e-5)
    c2 = (b2 - m2) * s2 + bt2

    dt = encoded_dists.T
    at = encoded_angles.T
    src = edge_index[0].reshape(_S, _NW, cpw_g, _CHG)
    dst = edge_index[1].reshape(_S, _NW, cpw_s, _CHS)

    nrb = 10
    xw = pl.pallas_call(
        _xw_kernel,
        grid=(nrb,),
        in_specs=[pl.BlockSpec((n // nrb, din), lambda i: (i, 0)),
                  pl.BlockSpec((din, dout), lambda i: (0, 0))],
        out_specs=pl.BlockSpec((n // nrb, dout), lambda i: (i, 0)),
        out_shape=jax.ShapeDtypeStruct((n, dout), jnp.float32),
    )(x, W1[:din])

    mesh = plsc.VectorSubcoreMesh(core_axis_name="c", subcore_axis_name="s")
    sc_params = pltpu.CompilerParams(use_tc_tiling_on_sc=False)

    def make_gather(si):
        @functools.partial(
            pl.kernel, mesh=mesh,
            out_type=jax.ShapeDtypeStruct((es, dout), jnp.float32),
            scratch_types=[pltpu.VMEM((cpw_g, _CHG), jnp.int32),
                           pltpu.VMEM((nbg, _CHG, dout), jnp.float32),
                           pltpu.SemaphoreType.DMA((nbg,)),
                           pltpu.SemaphoreType.DMA((nbg,))],
            compiler_params=sc_params,
        )
        def _gather_sc(table_hbm, idx_hbm, out_hbm, idx_all, rows, sem_g,
                       sem_s):
            wid = lax.axis_index("s") * _NC + lax.axis_index("c")
            base = wid * cpw_g
            pltpu.sync_copy(idx_hbm.at[si, wid], idx_all)
            for b in range(nbg):
                pltpu.async_copy(table_hbm.at[idx_all.at[b]], rows.at[b],
                                 sem_g.at[b])

            def group(gi, carry):
                for b in range(nbg):
                    j = gi * nbg + b
                    pltpu.make_async_copy(table_hbm.at[idx_all.at[j]],
                                          rows.at[b], sem_g.at[b]).wait()
                    pltpu.async_copy(
                        rows.at[b],
                        out_hbm.at[pl.ds((base + j) * _CHG, _CHG)],
                        sem_s.at[b])
                for b in range(nbg):
                    j = gi * nbg + b
                    jn = j + nbg

                    @pl.when(jn < cpw_g)
                    def _():
                        pltpu.make_async_copy(
                            rows.at[b],
                            out_hbm.at[pl.ds((base + j) * _CHG, _CHG)],
                            sem_s.at[b]).wait()
                        pltpu.async_copy(table_hbm.at[idx_all.at[jn]],
                                         rows.at[b], sem_g.at[b])
                return carry

            lax.fori_loop(0, ngr_g, group, 0)
            for b in range(nbg):
                j = (ngr_g - 1) * nbg + b
                pltpu.make_async_copy(
                    rows.at[b], out_hbm.at[pl.ds((base + j) * _CHG, _CHG)],
                    sem_s.at[b]).wait()

        return _gather_sc

    be = 3200
    eb = es // be
    assert eb * be == es

    def mlp_call(g_seg, si):
        off = si * eb

        def seg_map(i, o=off):
            return (o + i, 0)

        def tmap(i, o=off):
            return (0, o + i)

        zmap = lambda i: (0, 0)
        return pl.pallas_call(
            _mlp_kernel,
            grid=(eb,),
            in_specs=[pl.BlockSpec((be, dout), lambda i: (i, 0)),
                      pl.BlockSpec((dg, be), tmap),
                      pl.BlockSpec((dg, be), tmap),
                      pl.BlockSpec((dg, dout), zmap),
                      pl.BlockSpec((dg, dout), zmap),
                      pl.BlockSpec((1, dout), zmap),
                      pl.BlockSpec((1, dout), zmap),
                      pl.BlockSpec((dout, 2 * dout), zmap),
                      pl.BlockSpec((1, 2 * dout), zmap),
                      pl.BlockSpec((1, 2 * dout), zmap)],
            out_specs=pl.BlockSpec((be, dout), lambda i: (i, 0)),
            out_shape=jax.ShapeDtypeStruct((es, dout), jnp.float32),
        )(g_seg, dt, at, W1[din:din + dg].astype(bf),
          W1[din + dg:].astype(bf),
          s1[None], c1[None], W2.astype(bf), s2[None], c2[None])

    def make_scatter(si):
        @functools.partial(
            pl.kernel, mesh=mesh,
            out_type=jax.ShapeDtypeStruct((_NC, n, dout), jnp.float32),
            scratch_types=[pltpu.VMEM((cpw_s, _CHS), jnp.int32),
                           pltpu.VMEM((nbs, _CHS, dout), jnp.float32),
                           pltpu.VMEM_SHARED((n, dout), jnp.float32),
                           pltpu.SemaphoreType.DMA((nbs,)),
                           pltpu.SemaphoreType.DMA((nbs,))],
            compiler_params=sc_params,
        )
        def _scatter_sc(msg_hbm, idx_hbm, init_hbm, part_hbm, idx_all,
                        msg_v, acc, sem_l, sem_a):
            c = lax.axis_index("c")
            s = lax.axis_index("s")
            wid = s * _NC + c
            base = wid * cpw_s
            pltpu.sync_copy(idx_hbm.at[si, wid], idx_all)
            pltpu.sync_copy(init_hbm.at[c, pl.ds(s * rps, rps)],
                            acc.at[pl.ds(s * rps, rps)])
            plsc.subcore_barrier()
            for b in range(nbs):
                pltpu.async_copy(msg_hbm.at[pl.ds((base + b) * _CHS, _CHS)],
                                 msg_v.at[b], sem_l.at[b])

            def group(gi, carry):
                for b in range(nbs):
                    j = gi * nbs + b
                    pltpu.make_async_copy(
                        msg_hbm.at[pl.ds((base + j) * _CHS, _CHS)],
                        msg_v.at[b], sem_l.at[b]).wait()
                    pltpu.async_copy(msg_v.at[b], acc.at[idx_all.at[j]],
                                     sem_a.at[b], add=True)
                for b in range(nbs):
                    j = gi * nbs + b
                    jn = j + nbs

                    @pl.when(jn < cpw_s)
                    def _():
                        pltpu.make_async_copy(msg_v.at[b],
                                              acc.at[idx_all.at[j]],
                                              sem_a.at[b]).wait()
                        pltpu.async_copy(
                            msg_hbm.at[pl.ds((base + jn) * _CHS, _CHS)],
                            msg_v.at[b], sem_l.at[b])
                return carry

            lax.fori_loop(0, ngr_s, group, 0)
            for b in range(nbs):
                j = (ngr_s - 1) * nbs + b
                pltpu.make_async_copy(msg_v.at[b], acc.at[idx_all.at[j]],
                                      sem_a.at[b]).wait()
            plsc.subcore_barrier()
            pltpu.sync_copy(acc.at[pl.ds(s * rps, rps)],
                            part_hbm.at[c, pl.ds(s * rps, rps)])

        return _scatter_sc

    part = jnp.zeros((_NC, n, dout), jnp.float32)
    for si in range(_S):
        g_seg = make_gather(si)(xw, src)
        msg_seg = mlp_call(g_seg, si)
        part = make_scatter(si)(msg_seg, dst, part)

    out = pl.pallas_call(
        _psum_kernel,
        grid=(nrb,),
        in_specs=[pl.BlockSpec((_NC, n // nrb, dout), lambda i: (0, i, 0))],
        out_specs=pl.BlockSpec((n // nrb, dout), lambda i: (i, 0)),
        out_shape=jax.ShapeDtypeStruct((n, dout), jnp.float32),
    )(part)
    return out


# speedup vs baseline: 1.0481x; 1.0481x over previous
import functools

import jax
import jax.numpy as jnp
from jax import lax
from jax.experimental import pallas as pl
from jax.experimental.pallas import tpu as pltpu
from jax.experimental.pallas import tpu_sc as plsc

_NC = 2
_NS = 16
_NW = _NC * _NS

_CHG = 100
_CHS = 125
_S = 4


def _xw_kernel(x_ref, w_ref, o_ref):
    o_ref[...] = jnp.dot(x_ref[...], w_ref[...],
                         preferred_element_type=jnp.float32)


def _mlp_kernel(g_ref, dt_ref, at_ref, w1d_ref, w1a_ref, s1_ref, c1_ref,
                w2_ref, s2_ref, c2_ref, o_ref):
    d = g_ref.shape[1]
    bf = jnp.bfloat16
    cdim = (((0,), (0,)), ((), ()))
    t = (g_ref[...]
         + lax.dot_general(dt_ref[...].astype(bf), w1d_ref[...], cdim,
                           preferred_element_type=jnp.float32)
         + lax.dot_general(at_ref[...].astype(bf), w1a_ref[...], cdim,
                           preferred_element_type=jnp.float32))
    u = t * s1_ref[...] + c1_ref[...]
    u = u / (1.0 + jnp.exp(-u))
    v = jnp.dot(u.astype(bf), w2_ref[...], preferred_element_type=jnp.float32)
    v = v * s2_ref[...] + c2_ref[...]
    o_ref[...] = jnp.log1p(jnp.exp(v[:, d:])) / (1.0 + jnp.exp(-v[:, :d]))


def _psum_kernel(p0_ref, p1_ref, p2_ref, p3_ref, o_ref):
    o_ref[...] = ((p0_ref[0] + p0_ref[1]) + (p1_ref[0] + p1_ref[1])
                  + (p2_ref[0] + p2_ref[1]) + (p3_ref[0] + p3_ref[1]))


def kernel(x, edge_index, encoded_dists, encoded_angles, W1, b1, g1, bt1,
           m1, v1, W2, b2, g2, bt2, m2, v2):
    n, din = x.shape
    e = edge_index.shape[1]
    dout = W1.shape[1]
    dg = encoded_dists.shape[1]
    bf = jnp.bfloat16

    es = e // _S
    gch = es // _CHG
    cpw_g = gch // _NW
    nbg = 5
    ngr_g = cpw_g // nbg
    sch = es // _CHS
    cpw_s = sch // _NW
    nbs = 2
    ngr_s = cpw_s // nbs
    rps = n // _NS
    assert es * _S == e and gch * _CHG == es and sch * _CHS == es
    assert ngr_g * nbg == cpw_g and ngr_s * nbs == cpw_s and rps * _NS == n

    s1 = g1 * lax.rsqrt(v1 + 1e-5)
    c1 = (b1 - m1) * s1 + bt1
    s2 = g2 * lax.rsqrt(v2 + 1e-5)
    c2 = (b2 - m2) * s2 + bt2

    dt = encoded_dists.T
    at = encoded_angles.T
    src = edge_index[0].reshape(_S, _NW, cpw_g, _CHG)
    dst = edge_index[1].reshape(_S, _NW, cpw_s, _CHS)

    nrb = 10
    xw = pl.pallas_call(
        _xw_kernel,
        grid=(nrb,),
        in_specs=[pl.BlockSpec((n // nrb, din), lambda i: (i, 0)),
                  pl.BlockSpec((din, dout), lambda i: (0, 0))],
        out_specs=pl.BlockSpec((n // nrb, dout), lambda i: (i, 0)),
        out_shape=jax.ShapeDtypeStruct((n, dout), jnp.float32),
    )(x, W1[:din])

    mesh = plsc.VectorSubcoreMesh(core_axis_name="c", subcore_axis_name="s")
    sc_params = pltpu.CompilerParams(use_tc_tiling_on_sc=False)

    def make_gather(si):
        @functools.partial(
            pl.kernel, mesh=mesh,
            out_type=jax.ShapeDtypeStruct((es, dout), jnp.float32),
            scratch_types=[pltpu.VMEM((cpw_g, _CHG), jnp.int32),
                           pltpu.VMEM((nbg, _CHG, dout), jnp.float32),
                           pltpu.SemaphoreType.DMA((nbg,)),
                           pltpu.SemaphoreType.DMA((nbg,))],
            compiler_params=sc_params,
        )
        def _gather_sc(table_hbm, idx_hbm, out_hbm, idx_all, rows, sem_g,
                       sem_s):
            wid = lax.axis_index("s") * _NC + lax.axis_index("c")
            base = wid * cpw_g
            pltpu.sync_copy(idx_hbm.at[si, wid], idx_all)
            for b in range(nbg):
                pltpu.async_copy(table_hbm.at[idx_all.at[b]], rows.at[b],
                                 sem_g.at[b])

            def group(gi, carry):
                for b in range(nbg):
                    j = gi * nbg + b
                    pltpu.make_async_copy(table_hbm.at[idx_all.at[j]],
                                          rows.at[b], sem_g.at[b]).wait()
                    pltpu.async_copy(
                        rows.at[b],
                        out_hbm.at[pl.ds((base + j) * _CHG, _CHG)],
                        sem_s.at[b])
                for b in range(nbg):
                    j = gi * nbg + b
                    jn = j + nbg

                    @pl.when(jn < cpw_g)
                    def _():
                        pltpu.make_async_copy(
                            rows.at[b],
                            out_hbm.at[pl.ds((base + j) * _CHG, _CHG)],
                            sem_s.at[b]).wait()
                        pltpu.async_copy(table_hbm.at[idx_all.at[jn]],
                                         rows.at[b], sem_g.at[b])
                return carry

            lax.fori_loop(0, ngr_g, group, 0)
            for b in range(nbg):
                j = (ngr_g - 1) * nbg + b
                pltpu.make_async_copy(
                    rows.at[b], out_hbm.at[pl.ds((base + j) * _CHG, _CHG)],
                    sem_s.at[b]).wait()

        return _gather_sc

    be = 3200
    eb = es // be
    assert eb * be == es

    def mlp_call(g_seg, si):
        off = si * eb

        def seg_map(i, o=off):
            return (o + i, 0)

        def tmap(i, o=off):
            return (0, o + i)

        zmap = lambda i: (0, 0)
        return pl.pallas_call(
            _mlp_kernel,
            grid=(eb,),
            in_specs=[pl.BlockSpec((be, dout), lambda i: (i, 0)),
                      pl.BlockSpec((dg, be), tmap),
                      pl.BlockSpec((dg, be), tmap),
                      pl.BlockSpec((dg, dout), zmap),
                      pl.BlockSpec((dg, dout), zmap),
                      pl.BlockSpec((1, dout), zmap),
                      pl.BlockSpec((1, dout), zmap),
                      pl.BlockSpec((dout, 2 * dout), zmap),
                      pl.BlockSpec((1, 2 * dout), zmap),
                      pl.BlockSpec((1, 2 * dout), zmap)],
            out_specs=pl.BlockSpec((be, dout), lambda i: (i, 0)),
            out_shape=jax.ShapeDtypeStruct((es, dout), jnp.float32),
        )(g_seg, dt, at, W1[din:din + dg].astype(bf),
          W1[din + dg:].astype(bf),
          s1[None], c1[None], W2.astype(bf), s2[None], c2[None])

    def make_scatter(si):
        @functools.partial(
            pl.kernel, mesh=mesh,
            out_type=jax.ShapeDtypeStruct((_NC, n, dout), jnp.float32),
            scratch_types=[pltpu.VMEM((cpw_s, _CHS), jnp.int32),
                           pltpu.VMEM((nbs, _CHS, dout), jnp.float32),
                           pltpu.VMEM_SHARED((n, dout), jnp.float32),
                           pltpu.SemaphoreType.DMA((nbs,)),
                           pltpu.SemaphoreType.DMA((nbs,))],
            compiler_params=sc_params,
        )
        def _scatter_sc(msg_hbm, idx_hbm, zero_hbm, part_hbm, idx_all,
                        msg_v, acc, sem_l, sem_a):
            c = lax.axis_index("c")
            s = lax.axis_index("s")
            wid = s * _NC + c
            base = wid * cpw_s
            pltpu.sync_copy(idx_hbm.at[si, wid], idx_all)
            pltpu.sync_copy(zero_hbm.at[pl.ds(s * rps, rps)],
                            acc.at[pl.ds(s * rps, rps)])
            plsc.subcore_barrier()
            for b in range(nbs):
                pltpu.async_copy(msg_hbm.at[pl.ds((base + b) * _CHS, _CHS)],
                                 msg_v.at[b], sem_l.at[b])

            def group(gi, carry):
                for b in range(nbs):
                    j = gi * nbs + b
                    pltpu.make_async_copy(
                        msg_hbm.at[pl.ds((base + j) * _CHS, _CHS)],
                        msg_v.at[b], sem_l.at[b]).wait()
                    pltpu.async_copy(msg_v.at[b], acc.at[idx_all.at[j]],
                                     sem_a.at[b], add=True)
                for b in range(nbs):
                    j = gi * nbs + b
                    jn = j + nbs

                    @pl.when(jn < cpw_s)
                    def _():
                        pltpu.make_async_copy(msg_v.at[b],
                                              acc.at[idx_all.at[j]],
                                              sem_a.at[b]).wait()
                        pltpu.async_copy(
                            msg_hbm.at[pl.ds((base + jn) * _CHS, _CHS)],
                            msg_v.at[b], sem_l.at[b])
                return carry

            lax.fori_loop(0, ngr_s, group, 0)
            for b in range(nbs):
                j = (ngr_s - 1) * nbs + b
                pltpu.make_async_copy(msg_v.at[b], acc.at[idx_all.at[j]],
                                      sem_a.at[b]).wait()
            plsc.subcore_barrier()
            pltpu.sync_copy(acc.at[pl.ds(s * rps, rps)],
                            part_hbm.at[c, pl.ds(s * rps, rps)])

        return _scatter_sc

    zeros = jnp.zeros((n, dout), jnp.float32)

    parts = []
    for si in range(_S):
        g_seg = make_gather(si)(xw, src)
        msg_seg = mlp_call(g_seg, si)
        parts.append(make_scatter(si)(msg_seg, dst, zeros))

    out = pl.pallas_call(
        _psum_kernel,
        grid=(nrb,),
        in_specs=[pl.BlockSpec((_NC, n // nrb, dout), lambda i: (0, i, 0))
                  for _ in range(_S)],
        out_specs=pl.BlockSpec((n // nrb, dout), lambda i: (i, 0)),
        out_shape=jax.ShapeDtypeStruct((n, dout), jnp.float32),
    )(*parts)
    return out
